# baseline (speedup 1.0000x reference)
import jax
import jax.numpy as jnp
from jax import lax
from jax.experimental import pallas as pl
from jax.experimental.pallas import tpu as pltpu
from jax.experimental.pallas import tpu_sc as plsc

SEQ = 512
CLS_ID = 101
SEP_ID = 102
PAD_ID = 0
TRIM = SEQ - 2

NC = 2
NS = 16
NW = NC * NS
B = 1024
ROWS_PER_W = B // NW
HALF = ROWS_PER_W // 2


def _pack_body(tokens_hbm, lengths_hbm, word_hbm, mask_hbm,
               tok_v, word_v, mask_v, len_v,
               sem_in0, sem_in1, sem_o0, sem_o1, sem_o2, sem_o3):
    wid = lax.axis_index("s") * NC + lax.axis_index("c")
    base = wid * ROWS_PER_W

    cin0 = pltpu.async_copy(tokens_hbm.at[pl.ds(base, HALF)],
                            tok_v.at[pl.ds(0, HALF)], sem_in0)
    cin1 = pltpu.async_copy(tokens_hbm.at[pl.ds(base + HALF, HALF)],
                            tok_v.at[pl.ds(HALF, HALF)], sem_in1)
    pltpu.sync_copy(lengths_hbm.at[pl.ds(base, ROWS_PER_W)], len_v)

    iota16 = lax.iota(jnp.int32, 16)
    iota_m1 = iota16 - 1
    ones16 = jnp.full((16,), 1, jnp.int32)
    zeros16 = jnp.full((16,), 0, jnp.int32)

    def row_body(r, _):
        rvec = jnp.full((16,), r, jnp.int32)
        eff = jnp.minimum(plsc.load_gather(len_v, [rvec]), TRIM)
        eff1 = eff + 1
        eff_s = jnp.min(eff)
        kf = jnp.maximum((eff_s - 15) >> 4, 0) + 1
        kt = ((eff_s + 1) >> 4) + 1

        g0 = plsc.load_gather(tok_v, [rvec, jnp.maximum(iota_m1, 0)])
        sep0 = jnp.where(iota16 == eff1, jnp.int32(SEP_ID), jnp.int32(PAD_ID))
        w0 = jnp.where(iota16 == 0, jnp.int32(CLS_ID),
                       jnp.where(iota16 <= eff, g0, sep0))
        word_v[r, pl.ds(0, 16)] = w0
        mask_v[r, pl.ds(0, 16)] = jnp.where(iota16 <= eff1, ones16, zeros16)

        def full_body(k, _):
            i = k * 16
            g = plsc.load_gather(tok_v, [rvec, iota_m1 + i])
            word_v[r, pl.ds(i, 16)] = g
            mask_v[r, pl.ds(i, 16)] = ones16
            return 0

        def edge_body(k, _):
            i = k * 16
            p = iota16 + i
            g = plsc.load_gather(tok_v, [rvec, iota_m1 + i])
            word_v[r, pl.ds(i, 16)] = jnp.where(
                p <= eff, g,
                jnp.where(p == eff1, jnp.int32(SEP_ID), jnp.int32(PAD_ID)))
            mask_v[r, pl.ds(i, 16)] = jnp.where(p <= eff1, ones16, zeros16)
            return 0

        def tail_body(k, _):
            i = k * 16
            word_v[r, pl.ds(i, 16)] = zeros16
            mask_v[r, pl.ds(i, 16)] = zeros16
            return 0

        lax.fori_loop(1, kf, full_body, 0)
        lax.fori_loop(kf, kt, edge_body, 0)
        lax.fori_loop(kt, SEQ // 16, tail_body, 0)
        return 0

    cin0.wait()
    lax.fori_loop(0, HALF, row_body, 0)
    co0 = pltpu.async_copy(word_v.at[pl.ds(0, HALF)],
                           word_hbm.at[pl.ds(base, HALF)], sem_o0)
    co1 = pltpu.async_copy(mask_v.at[pl.ds(0, HALF)],
                           mask_hbm.at[pl.ds(base, HALF)], sem_o1)
    cin1.wait()
    lax.fori_loop(HALF, ROWS_PER_W, row_body, 0)
    co2 = pltpu.async_copy(word_v.at[pl.ds(HALF, HALF)],
                           word_hbm.at[pl.ds(base + HALF, HALF)], sem_o2)
    co3 = pltpu.async_copy(mask_v.at[pl.ds(HALF, HALF)],
                           mask_hbm.at[pl.ds(base + HALF, HALF)], sem_o3)
    co0.wait()
    co1.wait()
    co2.wait()
    co3.wait()


@jax.jit
def kernel(tokens, lengths):
    mesh = plsc.VectorSubcoreMesh(
        core_axis_name="c", subcore_axis_name="s",
        num_cores=NC, num_subcores=NS)
    out_word = jax.ShapeDtypeStruct((B, SEQ), jnp.int32)
    out_mask = jax.ShapeDtypeStruct((B, SEQ), jnp.int32)
    f = pl.kernel(
        _pack_body,
        out_type=(out_word, out_mask),
        mesh=mesh,
        scratch_types=[
            pltpu.VMEM((ROWS_PER_W, SEQ), jnp.int32),
            pltpu.VMEM((ROWS_PER_W, SEQ), jnp.int32),
            pltpu.VMEM((ROWS_PER_W, SEQ), jnp.int32),
            pltpu.VMEM((ROWS_PER_W,), jnp.int32),
            pltpu.SemaphoreType.DMA,
            pltpu.SemaphoreType.DMA,
            pltpu.SemaphoreType.DMA,
            pltpu.SemaphoreType.DMA,
            pltpu.SemaphoreType.DMA,
            pltpu.SemaphoreType.DMA,
        ],
        compiler_params=pltpu.CompilerParams(needs_layout_passes=False),
    )
    return f(tokens, lengths)


# speedup vs baseline: 1.0600x; 1.0600x over previous
import jax
import jax.numpy as jnp
from jax import lax
from jax.experimental import pallas as pl
from jax.experimental.pallas import tpu as pltpu
from jax.experimental.pallas import tpu_sc as plsc

SEQ = 512
CLS_ID = 101
SEP_ID = 102
PAD_ID = 0
TRIM = SEQ - 2

NC = 2
NS = 16
NW = NC * NS
B = 1024
ROWS_PER_W = B // NW
HALF = ROWS_PER_W // 2


def _words_body(tokens_hbm, lengths_hbm, word_hbm,
                tok_v, word_v, len_v,
                sem_in0, sem_in1, sem_o0, sem_o1):
    wid = lax.axis_index("s") * NC + lax.axis_index("c")
    base = wid * ROWS_PER_W

    cin0 = pltpu.async_copy(tokens_hbm.at[pl.ds(base, HALF)],
                            tok_v.at[pl.ds(0, HALF)], sem_in0)
    cin1 = pltpu.async_copy(tokens_hbm.at[pl.ds(base + HALF, HALF)],
                            tok_v.at[pl.ds(HALF, HALF)], sem_in1)
    pltpu.sync_copy(lengths_hbm.at[pl.ds(base, ROWS_PER_W)], len_v)

    iota16 = lax.iota(jnp.int32, 16)
    iota_m1 = iota16 - 1

    def row_body(r, _):
        rvec = jnp.full((16,), r, jnp.int32)
        eff = jnp.minimum(plsc.load_gather(len_v, [rvec]), TRIM)
        eff1 = eff + 1

        g0 = plsc.load_gather(tok_v, [rvec, jnp.maximum(iota_m1, 0)])
        sep0 = jnp.where(iota16 == eff1, jnp.int32(SEP_ID), jnp.int32(PAD_ID))
        word_v[r, pl.ds(0, 16)] = jnp.where(
            iota16 == 0, jnp.int32(CLS_ID),
            jnp.where(iota16 <= eff, g0, sep0))

        @plsc.parallel_loop(16, SEQ, step=16, unroll=8)
        def chunk_body(i):
            p = iota16 + i
            g = plsc.load_gather(tok_v, [rvec, iota_m1 + i])
            word_v[r, pl.ds(i, 16)] = jnp.where(
                p <= eff, g,
                jnp.where(p == eff1, jnp.int32(SEP_ID), jnp.int32(PAD_ID)))
        return 0

    cin0.wait()
    lax.fori_loop(0, HALF, row_body, 0)
    co0 = pltpu.async_copy(word_v.at[pl.ds(0, HALF)],
                           word_hbm.at[pl.ds(base, HALF)], sem_o0)
    cin1.wait()
    lax.fori_loop(HALF, ROWS_PER_W, row_body, 0)
    co1 = pltpu.async_copy(word_v.at[pl.ds(HALF, HALF)],
                           word_hbm.at[pl.ds(base + HALF, HALF)], sem_o1)
    co0.wait()
    co1.wait()


def _mask_body(len_ref, mask_ref):
    eff1 = jnp.minimum(len_ref[:, :], TRIM) + 1
    pos = lax.broadcasted_iota(jnp.int32, mask_ref.shape, 1)
    mask_ref[:, :] = jnp.where(pos <= eff1, jnp.int32(1), jnp.int32(0))


@jax.jit
def kernel(tokens, lengths):
    mesh = plsc.VectorSubcoreMesh(
        core_axis_name="c", subcore_axis_name="s",
        num_cores=NC, num_subcores=NS)
    words_fn = pl.kernel(
        _words_body,
        out_type=jax.ShapeDtypeStruct((B, SEQ), jnp.int32),
        mesh=mesh,
        scratch_types=[
            pltpu.VMEM((ROWS_PER_W, SEQ), jnp.int32),
            pltpu.VMEM((ROWS_PER_W, SEQ), jnp.int32),
            pltpu.VMEM((ROWS_PER_W,), jnp.int32),
            pltpu.SemaphoreType.DMA,
            pltpu.SemaphoreType.DMA,
            pltpu.SemaphoreType.DMA,
            pltpu.SemaphoreType.DMA,
        ],
        compiler_params=pltpu.CompilerParams(needs_layout_passes=False),
    )
    word_ids = words_fn(tokens, lengths)

    mask = pl.pallas_call(
        _mask_body,
        out_shape=jax.ShapeDtypeStruct((B, SEQ), jnp.int32),
        grid=(4,),
        in_specs=[pl.BlockSpec((B // 4, 1), lambda i: (i, 0))],
        out_specs=pl.BlockSpec((B // 4, SEQ), lambda i: (i, 0)),
    )(lengths.reshape(B, 1))
    return word_ids, mask
